# baseline (speedup 1.0000x reference)
import functools

import jax
import jax.numpy as jnp
from jax import lax
from jax.experimental import pallas as pl
from jax.experimental.pallas import tpu as pltpu
from jax.experimental.pallas import tpu_sc as plsc

N = 100000
E = 3200000
DC = 19
DP = 24
NC, NS = 2, 16
NW = NC * NS

CH = 80
KG = 25
RB = KG * CH
NR = E // RB
RG = NR // NW
RS = NR // NS
NP = 100352
RPT = NP // NS
DM = 24

_f32 = jnp.float32


def _lnk(v, g, b):
    m = jnp.mean(v, axis=-1, keepdims=True)
    var = jnp.mean((v - m) ** 2, axis=-1, keepdims=True)
    return (v - m) / jnp.sqrt(var + 1e-5) * g + b


def _row(a):
    return a.reshape(1, -1)


def _pad20(a):
    return jnp.pad(a, ((0, DP - a.shape[0]), (0, 0)))


def _padw(a, w):
    return jnp.pad(a, ((0, w - a.shape[0]), (0, 0)))


def _full_spec(a):
    return pl.BlockSpec(a.shape, lambda i: (0, 0))


def _encode(x, ws):
    BN = 4000
    grid = (N // BN,)

    def body(x_ref, *refs):
        (W0, b0, W1, b1, W2, b2, W3, b3, Wo, bo, Wi, bi, gi, bni) = refs[:14]
        sp_ref, xc_ref = refs[14], refs[15]
        h = x_ref[...]
        for W, b in ((W0, b0), (W1, b1), (W2, b2), (W3, b3)):
            h = jnp.tanh(h @ W[...] + b[...])
        sp = h @ Wo[...] + bo[...]
        hh = jnp.tanh(_lnk(sp @ Wi[...] + bi[...], gi[...], bni[...]))
        sp_ref[...] = sp
        xc_ref[...] = jnp.concatenate(
            [hh, sp, x_ref[...], jnp.zeros((BN, DP - 19), _f32)], axis=1)

    return pl.pallas_call(
        body,
        grid=grid,
        in_specs=[pl.BlockSpec((BN, 3), lambda i: (i, 0))] + [_full_spec(w) for w in ws],
        out_specs=[pl.BlockSpec((BN, 8), lambda i: (i, 0)),
                   pl.BlockSpec((BN, DP), lambda i: (i, 0))],
        out_shape=[jax.ShapeDtypeStruct((N, 8), _f32),
                   jax.ShapeDtypeStruct((N, DP), _f32)],
    )(x, *ws)


def _edge_pass(gs, gd, ws, final):
    BE = 5120
    grid = (E // BE,)

    def body(gs_ref, gd_ref, *refs):
        (W0a, W0b, b0, g0, bn0, W1, b1, g1, bn1,
         W2, b2, g2, bn2, W3, b3) = refs[:15]
        gsv = gs_ref[...]
        gdv = gd_ref[...]
        hh = gsv @ W0a[...] + gdv @ W0b[...] + b0[...]
        hh = jnp.tanh(_lnk(hh, g0[...], bn0[...]))
        hh = jnp.tanh(_lnk(hh @ W1[...] + b1[...], g1[...], bn1[...]))
        hh = jnp.tanh(_lnk(hh @ W2[...] + b2[...], g2[...], bn2[...]))
        lg = hh @ W3[...] + b3[...]
        if final:
            refs[15][...] = lg[:, 0]
        else:
            refs[15][...] = jax.nn.sigmoid(lg)[:, 0]

    out_specs = [pl.BlockSpec((BE,), lambda i: (i,))]
    out_shape = [jax.ShapeDtypeStruct((E,), _f32)]

    return pl.pallas_call(
        body,
        grid=grid,
        in_specs=[pl.BlockSpec((BE, DP), lambda i: (i, 0))] * 2 + [_full_spec(w) for w in ws],
        out_specs=out_specs,
        out_shape=out_shape,
    )(gs, gd, *ws)


def _node_pass(mi, mo, xc, sp, x, ws):
    BN = 4000
    grid = (N // BN,)

    def body(mi_ref, mo_ref, xc_ref, sp_ref, x_ref, *refs):
        (W0a, W0b, W0c, b0, g0, bn0, W1, b1, g1, bn1,
         W2, b2, g2, bn2, W3, b3) = refs[:16]
        out_ref = refs[16]
        xcv = xc_ref[...]
        hh = mi_ref[...] @ W0a[...] + mo_ref[...] @ W0b[...] + xcv @ W0c[...] + b0[...]
        hh = jnp.tanh(_lnk(hh, g0[...], bn0[...]))
        hh = jnp.tanh(_lnk(hh @ W1[...] + b1[...], g1[...], bn1[...]))
        hh = jnp.tanh(_lnk(hh @ W2[...] + b2[...], g2[...], bn2[...]))
        xn = hh @ W3[...] + b3[...]
        out_ref[...] = xcv + jnp.concatenate(
            [xn, sp_ref[...], x_ref[...], jnp.zeros((BN, DP - 19), _f32)], axis=1)

    return pl.pallas_call(
        body,
        grid=grid,
        in_specs=[pl.BlockSpec((BN, DM), lambda i: (i, 0)),
                  pl.BlockSpec((BN, DM), lambda i: (i, 0)),
                  pl.BlockSpec((BN, DP), lambda i: (i, 0)),
                  pl.BlockSpec((BN, 8), lambda i: (i, 0)),
                  pl.BlockSpec((BN, 3), lambda i: (i, 0))] + [_full_spec(w) for w in ws],
        out_specs=[pl.BlockSpec((BN, DP), lambda i: (i, 0))],
        out_shape=[jax.ShapeDtypeStruct((N, DP), _f32)],
    )(mi, mo, xc, sp, x, *ws)


def _sc_mesh():
    return plsc.VectorSubcoreMesh(core_axis_name="c", subcore_axis_name="s",
                                  num_cores=NC, num_subcores=NS)


@functools.partial(
    pl.kernel,
    out_type=(jax.ShapeDtypeStruct((E, DP), _f32),
              jax.ShapeDtypeStruct((E, DP), _f32)),
    mesh=_sc_mesh(),
    compiler_params=pltpu.CompilerParams(use_tc_tiling_on_sc=False),
    scratch_types=[pltpu.SemaphoreType.DMA, pltpu.SemaphoreType.DMA]
    + [pltpu.VMEM((CH,), jnp.int32) for _ in range(KG)]
    + [pltpu.VMEM((CH, DP), _f32) for _ in range(KG)],
)
def _gather_k(table_hbm, s3_hbm, d3_hbm, gs_hbm, gd_hbm, sem, sem2, *bufs):
    idx_bufs = bufs[:KG]
    row_bufs = bufs[KG:]
    w = lax.axis_index("s") * NC + lax.axis_index("c")
    for idx_hbm, out_hbm in ((s3_hbm, gs_hbm), (d3_hbm, gd_hbm)):
        def round_body(r, carry, idx_hbm=idx_hbm, out_hbm=out_hbm):
            rid = w * RG + r
            idescs = [pltpu.async_copy(idx_hbm.at[rid, k], idx_bufs[k], sem2)
                      for k in range(KG)]
            for dsc in idescs:
                dsc.wait()
            descs = [pltpu.async_copy(table_hbm.at[idx_bufs[k]],
                                      row_bufs[k], sem)
                     for k in range(KG)]
            for dsc in descs:
                dsc.wait()
            odescs = [pltpu.async_copy(row_bufs[k],
                                       out_hbm.at[pl.ds(rid * RB + k * CH, CH)],
                                       sem2)
                      for k in range(KG)]
            for dsc in odescs:
                dsc.wait()
            return carry
        lax.fori_loop(0, RG, round_body, 0)


@functools.partial(
    pl.kernel,
    out_type=(jax.ShapeDtypeStruct((NP, DM), _f32),
              jax.ShapeDtypeStruct((NP, DM), _f32)),
    mesh=_sc_mesh(),
    compiler_params=pltpu.CompilerParams(use_tc_tiling_on_sc=False,
                                         needs_layout_passes=False),
    scratch_types=[pltpu.VMEM((KG, CH), jnp.int32),
                   pltpu.VMEM((KG * CH, 8), _f32),
                   pltpu.VMEM((KG * CH,), _f32),
                   pltpu.SemaphoreType.DMA,
                   pltpu.VMEM_SHARED((NP, 8), _f32)],
)
def _scatter_k(gs_hbm, gd_hbm, e_hbm, d3_hbm, s3_hbm, z_hbm, mi_hbm, mo_hbm,
               idx_v, rows_v, e_v, sem, acc):
    c = lax.axis_index("c")
    t = lax.axis_index("s")

    def run(rows_hbm, idx_hbm, out_hbm):
        for h in range(3):
            pltpu.sync_copy(z_hbm.at[pl.ds(t * RPT, RPT)],
                            acc.at[pl.ds(t * RPT, RPT)])
            plsc.subcore_barrier()

            def round_body(r, carry, h=h):
                rid = t * RS + r
                pltpu.sync_copy(idx_hbm.at[rid], idx_v)
                pltpu.sync_copy(
                    rows_hbm.at[pl.ds(rid * RB, RB), pl.ds(h * 8, 8)], rows_v)
                pltpu.sync_copy(e_hbm.at[pl.ds(rid * RB, RB)], e_v)

                def grp(g, cc):
                    rowi = g * 16 + lax.iota(jnp.int32, 16)
                    ev = e_v[pl.ds(g * 16, 16)]
                    for j in range(8):
                        colj = jnp.full((16,), j, jnp.int32)
                        vals = plsc.load_gather(rows_v, [rowi, colj])
                        plsc.store_scatter(rows_v, [rowi, colj], vals * ev)
                    return cc
                lax.fori_loop(0, RB // 16, grp, 0)

                descs = [pltpu.async_copy(rows_v.at[pl.ds(k * CH, CH)],
                                          acc.at[idx_v.at[k]], sem, add=True)
                         for k in range(KG)]
                for dsc in descs:
                    dsc.wait()
                return carry
            lax.fori_loop(0, RS, round_body, 0)
            plsc.subcore_barrier()
            pltpu.sync_copy(acc.at[pl.ds(t * RPT, RPT)],
                            out_hbm.at[pl.ds(t * RPT, RPT), pl.ds(h * 8, 8)])
            plsc.subcore_barrier()

    @pl.when(c == 0)
    def _():
        run(gs_hbm, d3_hbm, mi_hbm)

    @pl.when(c == 1)
    def _():
        run(gd_hbm, s3_hbm, mo_hbm)


def kernel(x, edge_index, params):
    p = params
    el = p['emb_layers']
    eo = p['emb_out']
    ip = p['input_net']
    en = p['edge_net']
    nn = p['node_net']

    enc_ws = [el[0]['W'], _row(el[0]['b']), el[1]['W'], _row(el[1]['b']),
              el[2]['W'], _row(el[2]['b']), el[3]['W'], _row(el[3]['b']),
              eo['W'], _row(eo['b']),
              ip['W'], _row(ip['b']), _row(ip['g']), _row(ip['bn'])]
    edge_ws = [_pad20(en[0]['W'][:DC]), _pad20(en[0]['W'][DC:]),
               _row(en[0]['b']), _row(en[0]['g']), _row(en[0]['bn']),
               en[1]['W'], _row(en[1]['b']), _row(en[1]['g']), _row(en[1]['bn']),
               en[2]['W'], _row(en[2]['b']), _row(en[2]['g']), _row(en[2]['bn']),
               en[3]['W'], _row(en[3]['b'])]
    node_ws = [_padw(nn[0]['W'][:DC], DM), _padw(nn[0]['W'][DC:2 * DC], DM),
               _pad20(nn[0]['W'][2 * DC:]),
               _row(nn[0]['b']), _row(nn[0]['g']), _row(nn[0]['bn']),
               nn[1]['W'], _row(nn[1]['b']), _row(nn[1]['g']), _row(nn[1]['bn']),
               nn[2]['W'], _row(nn[2]['b']), _row(nn[2]['g']), _row(nn[2]['bn']),
               nn[3]['W'], _row(nn[3]['b'])]

    s3 = edge_index[0].reshape(NR, KG, CH)
    d3 = edge_index[1].reshape(NR, KG, CH)
    zeros_np = jnp.zeros((NP, 8), _f32)

    spatial, xc = _encode(x, enc_ws)
    for _ in range(3):
        gs, gd = _gather_k(xc, s3, d3)
        e = _edge_pass(gs, gd, edge_ws, final=False)[0]
        mi, mo = _scatter_k(gs, gd, e, d3, s3, zeros_np)
        xc = _node_pass(mi, mo, xc, spatial, x, node_ws)[0]
    gs, gd = _gather_k(xc, s3, d3)
    logits = _edge_pass(gs, gd, edge_ws, final=True)[0]

    return logits, spatial, edge_index, edge_index.shape[1] / x.shape[0]


# speedup vs baseline: 2.3872x; 2.3872x over previous
import functools

import jax
import jax.numpy as jnp
from jax import lax
from jax.experimental import pallas as pl
from jax.experimental.pallas import tpu as pltpu
from jax.experimental.pallas import tpu_sc as plsc

N = 100000
E = 3200000
DC = 19
DP = 24
NC, NS = 2, 16
NW = NC * NS

CH = 80
KG = 25
RB = KG * CH
NR = E // RB
RG = NR // NW
RS = NR // NS
NP = 100352
RPT = NP // NS
DM = 24

_f32 = jnp.float32


def _lnk(v, g, b):
    m = jnp.mean(v, axis=-1, keepdims=True)
    var = jnp.mean((v - m) ** 2, axis=-1, keepdims=True)
    return (v - m) / jnp.sqrt(var + 1e-5) * g + b


def _row(a):
    return a.reshape(1, -1)


def _pad20(a):
    return jnp.pad(a, ((0, DP - a.shape[0]), (0, 0)))


def _padw(a, w):
    return jnp.pad(a, ((0, w - a.shape[0]), (0, 0)))


def _full_spec(a):
    return pl.BlockSpec(a.shape, lambda i: (0, 0))


def _encode(x, ws):
    BN = 4000
    grid = (N // BN,)

    def body(x_ref, *refs):
        (W0, b0, W1, b1, W2, b2, W3, b3, Wo, bo, Wi, bi, gi, bni) = refs[:14]
        sp_ref, xc_ref = refs[14], refs[15]
        h = x_ref[...]
        for W, b in ((W0, b0), (W1, b1), (W2, b2), (W3, b3)):
            h = jnp.tanh(h @ W[...] + b[...])
        sp = h @ Wo[...] + bo[...]
        hh = jnp.tanh(_lnk(sp @ Wi[...] + bi[...], gi[...], bni[...]))
        sp_ref[...] = sp
        xc_ref[...] = jnp.concatenate(
            [hh, sp, x_ref[...], jnp.zeros((BN, DP - 19), _f32)], axis=1)

    return pl.pallas_call(
        body,
        grid=grid,
        in_specs=[pl.BlockSpec((BN, 3), lambda i: (i, 0))] + [_full_spec(w) for w in ws],
        out_specs=[pl.BlockSpec((BN, 8), lambda i: (i, 0)),
                   pl.BlockSpec((BN, DP), lambda i: (i, 0))],
        out_shape=[jax.ShapeDtypeStruct((N, 8), _f32),
                   jax.ShapeDtypeStruct((N, DP), _f32)],
    )(x, *ws)


E16 = E // 16


def _lnp(v, g, b, M):
    m = v @ M
    var = (v * v) @ M - m * m
    return (v - m) / jnp.sqrt(var + 1e-5) * g + b


def _edge_pass(gsp, gdp, ws, final):
    BE2 = 1000
    grid = (E16 // BE2,)

    def body(gs_ref, gd_ref, *refs):
        (Wa, Wb, b0, g0, bn0, M, W1, b1, g1, bn1,
         W2, b2, g2, bn2, W3, b3) = refs[:16]
        Mv = M[...]
        hh = gs_ref[...] @ Wa[...] + gd_ref[...] @ Wb[...] + b0[...]
        hh = jnp.tanh(_lnp(hh, g0[...], bn0[...], Mv))
        hh = jnp.tanh(_lnp(hh @ W1[...] + b1[...], g1[...], bn1[...], Mv))
        hh = jnp.tanh(_lnp(hh @ W2[...] + b2[...], g2[...], bn2[...], Mv))
        lg = hh @ W3[...] + b3[...]
        if final:
            refs[16][...] = lg
        else:
            refs[16][...] = jax.nn.sigmoid(lg)

    return pl.pallas_call(
        body,
        grid=grid,
        in_specs=[pl.BlockSpec((BE2, 384), lambda i: (i, 0))] * 2
        + [_full_spec(w) for w in ws],
        out_specs=[pl.BlockSpec((BE2, 16), lambda i: (i, 0))],
        out_shape=[jax.ShapeDtypeStruct((E16, 16), _f32)],
    )(gsp, gdp, *ws)


def _node_pass(mi, mo, xc, sp, x, ws):
    BN = 4000
    grid = (N // BN,)

    def body(mi_ref, mo_ref, xc_ref, sp_ref, x_ref, *refs):
        (W0a, W0b, W0c, b0, g0, bn0, W1, b1, g1, bn1,
         W2, b2, g2, bn2, W3, b3) = refs[:16]
        out_ref = refs[16]
        xcv = xc_ref[...]
        hh = mi_ref[...] @ W0a[...] + mo_ref[...] @ W0b[...] + xcv @ W0c[...] + b0[...]
        hh = jnp.tanh(_lnk(hh, g0[...], bn0[...]))
        hh = jnp.tanh(_lnk(hh @ W1[...] + b1[...], g1[...], bn1[...]))
        hh = jnp.tanh(_lnk(hh @ W2[...] + b2[...], g2[...], bn2[...]))
        xn = hh @ W3[...] + b3[...]
        out_ref[...] = xcv + jnp.concatenate(
            [xn, sp_ref[...], x_ref[...], jnp.zeros((BN, DP - 19), _f32)], axis=1)

    return pl.pallas_call(
        body,
        grid=grid,
        in_specs=[pl.BlockSpec((BN, DM), lambda i: (i, 0)),
                  pl.BlockSpec((BN, DM), lambda i: (i, 0)),
                  pl.BlockSpec((BN, DP), lambda i: (i, 0)),
                  pl.BlockSpec((BN, 8), lambda i: (i, 0)),
                  pl.BlockSpec((BN, 3), lambda i: (i, 0))] + [_full_spec(w) for w in ws],
        out_specs=[pl.BlockSpec((BN, DP), lambda i: (i, 0))],
        out_shape=[jax.ShapeDtypeStruct((N, DP), _f32)],
    )(mi, mo, xc, sp, x, *ws)


def _sc_mesh():
    return plsc.VectorSubcoreMesh(core_axis_name="c", subcore_axis_name="s",
                                  num_cores=NC, num_subcores=NS)


@functools.partial(
    pl.kernel,
    out_type=(jax.ShapeDtypeStruct((E, DP), _f32),
              jax.ShapeDtypeStruct((E, DP), _f32)),
    mesh=_sc_mesh(),
    compiler_params=pltpu.CompilerParams(use_tc_tiling_on_sc=False),
    scratch_types=[pltpu.SemaphoreType.DMA, pltpu.SemaphoreType.DMA]
    + [pltpu.VMEM((CH,), jnp.int32) for _ in range(KG)]
    + [pltpu.VMEM((CH, DP), _f32) for _ in range(KG)],
)
def _gather_k(table_hbm, s3_hbm, d3_hbm, gs_hbm, gd_hbm, sem, sem2, *bufs):
    idx_bufs = bufs[:KG]
    row_bufs = bufs[KG:]
    w = lax.axis_index("s") * NC + lax.axis_index("c")
    for idx_hbm, out_hbm in ((s3_hbm, gs_hbm), (d3_hbm, gd_hbm)):
        def round_body(r, carry, idx_hbm=idx_hbm, out_hbm=out_hbm):
            rid = w * RG + r
            idescs = [pltpu.async_copy(idx_hbm.at[rid, k], idx_bufs[k], sem2)
                      for k in range(KG)]
            for dsc in idescs:
                dsc.wait()
            descs = [pltpu.async_copy(table_hbm.at[idx_bufs[k]],
                                      row_bufs[k], sem)
                     for k in range(KG)]
            for dsc in descs:
                dsc.wait()
            odescs = [pltpu.async_copy(row_bufs[k],
                                       out_hbm.at[pl.ds(rid * RB + k * CH, CH)],
                                       sem2)
                      for k in range(KG)]
            for dsc in odescs:
                dsc.wait()
            return carry
        lax.fori_loop(0, RG, round_body, 0)


@functools.partial(
    pl.kernel,
    out_type=(jax.ShapeDtypeStruct((NP, DM), _f32),
              jax.ShapeDtypeStruct((NP, DM), _f32)),
    mesh=_sc_mesh(),
    compiler_params=pltpu.CompilerParams(use_tc_tiling_on_sc=False,
                                         needs_layout_passes=False),
    scratch_types=[pltpu.VMEM((KG, CH), jnp.int32),
                   pltpu.VMEM((KG * CH, 8), _f32),
                   pltpu.VMEM((KG * CH // 16, 16), _f32),
                   pltpu.SemaphoreType.DMA,
                   pltpu.VMEM_SHARED((NP, 8), _f32)],
)
def _scatter_k(gs_hbm, gd_hbm, e_hbm, d3_hbm, s3_hbm, z_hbm, mi_hbm, mo_hbm,
               idx_v, rows_v, e_v, sem, acc):
    c = lax.axis_index("c")
    t = lax.axis_index("s")

    def run(rows_hbm, idx_hbm, out_hbm):
        for h in range(3):
            pltpu.sync_copy(z_hbm.at[pl.ds(t * RPT, RPT)],
                            acc.at[pl.ds(t * RPT, RPT)])
            plsc.subcore_barrier()

            def round_body(r, carry, h=h):
                rid = t * RS + r
                pltpu.sync_copy(idx_hbm.at[rid], idx_v)
                pltpu.sync_copy(
                    rows_hbm.at[pl.ds(rid * RB, RB), pl.ds(h * 8, 8)], rows_v)
                pltpu.sync_copy(e_hbm.at[pl.ds(rid * (RB // 16), RB // 16)], e_v)

                def grp(g, cc):
                    rowi = g * 16 + lax.iota(jnp.int32, 16)
                    ev = e_v[g]
                    for j in range(8):
                        colj = jnp.full((16,), j, jnp.int32)
                        vals = plsc.load_gather(rows_v, [rowi, colj])
                        plsc.store_scatter(rows_v, [rowi, colj], vals * ev)
                    return cc
                lax.fori_loop(0, RB // 16, grp, 0)

                descs = [pltpu.async_copy(rows_v.at[pl.ds(k * CH, CH)],
                                          acc.at[idx_v.at[k]], sem, add=True)
                         for k in range(KG)]
                for dsc in descs:
                    dsc.wait()
                return carry
            lax.fori_loop(0, RS, round_body, 0)
            plsc.subcore_barrier()
            pltpu.sync_copy(acc.at[pl.ds(t * RPT, RPT)],
                            out_hbm.at[pl.ds(t * RPT, RPT), pl.ds(h * 8, 8)])
            plsc.subcore_barrier()

    @pl.when(c == 0)
    def _():
        run(gs_hbm, d3_hbm, mi_hbm)

    @pl.when(c == 1)
    def _():
        run(gd_hbm, s3_hbm, mo_hbm)


def kernel(x, edge_index, params):
    p = params
    el = p['emb_layers']
    eo = p['emb_out']
    ip = p['input_net']
    en = p['edge_net']
    nn = p['node_net']

    enc_ws = [el[0]['W'], _row(el[0]['b']), el[1]['W'], _row(el[1]['b']),
              el[2]['W'], _row(el[2]['b']), el[3]['W'], _row(el[3]['b']),
              eo['W'], _row(eo['b']),
              ip['W'], _row(ip['b']), _row(ip['g']), _row(ip['bn'])]
    I16 = jnp.eye(16, dtype=_f32)
    t16 = lambda v: jnp.tile(_row(v), (1, 16))
    M11 = jnp.kron(I16, jnp.full((11, 11), 1.0 / 11.0, _f32))
    edge_ws = [jnp.kron(I16, _pad20(en[0]['W'][:DC])),
               jnp.kron(I16, _pad20(en[0]['W'][DC:])),
               t16(en[0]['b']), t16(en[0]['g']), t16(en[0]['bn']), M11,
               jnp.kron(I16, en[1]['W']), t16(en[1]['b']), t16(en[1]['g']), t16(en[1]['bn']),
               jnp.kron(I16, en[2]['W']), t16(en[2]['b']), t16(en[2]['g']), t16(en[2]['bn']),
               jnp.kron(I16, en[3]['W']), t16(en[3]['b'])]
    node_ws = [_padw(nn[0]['W'][:DC], DM), _padw(nn[0]['W'][DC:2 * DC], DM),
               _pad20(nn[0]['W'][2 * DC:]),
               _row(nn[0]['b']), _row(nn[0]['g']), _row(nn[0]['bn']),
               nn[1]['W'], _row(nn[1]['b']), _row(nn[1]['g']), _row(nn[1]['bn']),
               nn[2]['W'], _row(nn[2]['b']), _row(nn[2]['g']), _row(nn[2]['bn']),
               nn[3]['W'], _row(nn[3]['b'])]

    s3 = edge_index[0].reshape(NR, KG, CH)
    d3 = edge_index[1].reshape(NR, KG, CH)
    zeros_np = jnp.zeros((NP, 8), _f32)

    spatial, xc = _encode(x, enc_ws)
    for _ in range(3):
        gs, gd = _gather_k(xc, s3, d3)
        gsp = gs.reshape(E16, 384)
        gdp = gd.reshape(E16, 384)
        e = _edge_pass(gsp, gdp, edge_ws, final=False)[0]
        mi, mo = _scatter_k(gs, gd, e, d3, s3, zeros_np)
        xc = _node_pass(mi, mo, xc, spatial, x, node_ws)[0]
    gs, gd = _gather_k(xc, s3, d3)
    logits = _edge_pass(gs.reshape(E16, 384), gd.reshape(E16, 384),
                        edge_ws, final=True)[0].reshape(E)

    return logits, spatial, edge_index, edge_index.shape[1] / x.shape[0]


# speedup vs baseline: 2.4250x; 1.0158x over previous
import functools

import jax
import jax.numpy as jnp
from jax import lax
from jax.experimental import pallas as pl
from jax.experimental.pallas import tpu as pltpu
from jax.experimental.pallas import tpu_sc as plsc

N = 100000
E = 3200000
DC = 19
DP = 24
NC, NS = 2, 16
NW = NC * NS

CH = 80
KG = 25
RB = KG * CH
NR = E // RB
RG = NR // NW
RS = NR // NS
NP = 100352
RPT = NP // NS
DM = 24

_f32 = jnp.float32


def _lnk(v, g, b):
    m = jnp.mean(v, axis=-1, keepdims=True)
    var = jnp.mean((v - m) ** 2, axis=-1, keepdims=True)
    return (v - m) / jnp.sqrt(var + 1e-5) * g + b


def _row(a):
    return a.reshape(1, -1)


def _pad20(a):
    return jnp.pad(a, ((0, DP - a.shape[0]), (0, 0)))


def _padw(a, w):
    return jnp.pad(a, ((0, w - a.shape[0]), (0, 0)))


def _full_spec(a):
    return pl.BlockSpec(a.shape, lambda i: (0, 0))


def _encode(x, ws):
    BN = 4000
    grid = (N // BN,)

    def body(x_ref, *refs):
        (W0, b0, W1, b1, W2, b2, W3, b3, Wo, bo, Wi, bi, gi, bni) = refs[:14]
        sp_ref, xc_ref = refs[14], refs[15]
        h = x_ref[...]
        for W, b in ((W0, b0), (W1, b1), (W2, b2), (W3, b3)):
            h = jnp.tanh(h @ W[...] + b[...])
        sp = h @ Wo[...] + bo[...]
        hh = jnp.tanh(_lnk(sp @ Wi[...] + bi[...], gi[...], bni[...]))
        sp_ref[...] = sp
        xc_ref[...] = jnp.concatenate(
            [hh, sp, x_ref[...], jnp.zeros((BN, DP - 19), _f32)], axis=1)

    return pl.pallas_call(
        body,
        grid=grid,
        in_specs=[pl.BlockSpec((BN, 3), lambda i: (i, 0))] + [_full_spec(w) for w in ws],
        out_specs=[pl.BlockSpec((BN, 8), lambda i: (i, 0)),
                   pl.BlockSpec((BN, DP), lambda i: (i, 0))],
        out_shape=[jax.ShapeDtypeStruct((N, 8), _f32),
                   jax.ShapeDtypeStruct((N, DP), _f32)],
    )(x, *ws)


E16 = E // 16


def _lnp(v, g, b, M):
    m = v @ M
    var = (v * v) @ M - m * m
    return (v - m) / jnp.sqrt(var + 1e-5) * g + b


def _edge_pass(gsp, gdp, ws, final):
    BE2 = 1000
    grid = (E16 // BE2,)

    def body(gs_ref, gd_ref, *refs):
        (Wa, Wb, b0, g0, bn0, M, W1, b1, g1, bn1,
         W2, b2, g2, bn2, W3, b3) = refs[:16]
        Mv = M[...]
        hh = gs_ref[...] @ Wa[...] + gd_ref[...] @ Wb[...] + b0[...]
        hh = jnp.tanh(_lnp(hh, g0[...], bn0[...], Mv))
        hh = jnp.tanh(_lnp(hh @ W1[...] + b1[...], g1[...], bn1[...], Mv))
        hh = jnp.tanh(_lnp(hh @ W2[...] + b2[...], g2[...], bn2[...], Mv))
        lg = hh @ W3[...] + b3[...]
        if final:
            refs[16][...] = lg
        else:
            refs[16][...] = jax.nn.sigmoid(lg)

    return pl.pallas_call(
        body,
        grid=grid,
        in_specs=[pl.BlockSpec((BE2, 384), lambda i: (i, 0))] * 2
        + [_full_spec(w) for w in ws],
        out_specs=[pl.BlockSpec((BE2, 16), lambda i: (i, 0))],
        out_shape=[jax.ShapeDtypeStruct((E16, 16), _f32)],
    )(gsp, gdp, *ws)


def _node_pass(mi, mo, xc, sp, x, ws):
    BN = 4000
    grid = (N // BN,)

    def body(mi_ref, mo_ref, xc_ref, sp_ref, x_ref, *refs):
        (W0a, W0b, W0c, b0, g0, bn0, W1, b1, g1, bn1,
         W2, b2, g2, bn2, W3, b3) = refs[:16]
        out_ref = refs[16]
        xcv = xc_ref[...]
        hh = mi_ref[...] @ W0a[...] + mo_ref[...] @ W0b[...] + xcv @ W0c[...] + b0[...]
        hh = jnp.tanh(_lnk(hh, g0[...], bn0[...]))
        hh = jnp.tanh(_lnk(hh @ W1[...] + b1[...], g1[...], bn1[...]))
        hh = jnp.tanh(_lnk(hh @ W2[...] + b2[...], g2[...], bn2[...]))
        xn = hh @ W3[...] + b3[...]
        out_ref[...] = xcv + jnp.concatenate(
            [xn, sp_ref[...], x_ref[...], jnp.zeros((BN, DP - 19), _f32)], axis=1)

    return pl.pallas_call(
        body,
        grid=grid,
        in_specs=[pl.BlockSpec((BN, DM), lambda i: (i, 0)),
                  pl.BlockSpec((BN, DM), lambda i: (i, 0)),
                  pl.BlockSpec((BN, DP), lambda i: (i, 0)),
                  pl.BlockSpec((BN, 8), lambda i: (i, 0)),
                  pl.BlockSpec((BN, 3), lambda i: (i, 0))] + [_full_spec(w) for w in ws],
        out_specs=[pl.BlockSpec((BN, DP), lambda i: (i, 0))],
        out_shape=[jax.ShapeDtypeStruct((N, DP), _f32)],
    )(mi, mo, xc, sp, x, *ws)


def _sc_mesh():
    return plsc.VectorSubcoreMesh(core_axis_name="c", subcore_axis_name="s",
                                  num_cores=NC, num_subcores=NS)


@functools.partial(
    pl.kernel,
    out_type=(jax.ShapeDtypeStruct((E, DP), _f32),
              jax.ShapeDtypeStruct((E, DP), _f32)),
    mesh=_sc_mesh(),
    compiler_params=pltpu.CompilerParams(use_tc_tiling_on_sc=False),
    scratch_types=[pltpu.SemaphoreType.DMA, pltpu.SemaphoreType.DMA,
                   pltpu.SemaphoreType.DMA]
    + [pltpu.VMEM((CH,), jnp.int32) for _ in range(2 * KG)]
    + [pltpu.VMEM((CH, DP), _f32) for _ in range(2 * KG)],
)
def _gather_k(table_hbm, s3_hbm, d3_hbm, gs_hbm, gd_hbm, sem, sem2, sem3,
              *bufs):
    idx_sets = (bufs[:KG], bufs[KG:2 * KG])
    row_sets = (bufs[2 * KG:3 * KG], bufs[3 * KG:])
    w = lax.axis_index("s") * NC + lax.axis_index("c")

    for idx_hbm, out_hbm in ((s3_hbm, gs_hbm), (d3_hbm, gd_hbm)):
        def fire_idx(rr, st):
            for k in range(KG):
                pltpu.async_copy(idx_hbm.at[w * RG + rr, k], idx_sets[st][k],
                                 sem2)

        def drain_idx(rr, st):
            for k in range(KG):
                pltpu.make_async_copy(idx_hbm.at[w * RG + rr, k],
                                      idx_sets[st][k], sem2).wait()

        def fire_gathers(st):
            for k in range(KG):
                pltpu.async_copy(table_hbm.at[idx_sets[st][k]],
                                 row_sets[st][k], sem)

        def drain_gathers(st):
            for k in range(KG):
                pltpu.make_async_copy(table_hbm.at[idx_sets[st][k]],
                                      row_sets[st][k], sem).wait()

        def fire_stores(rr, st):
            for k in range(KG):
                pltpu.async_copy(
                    row_sets[st][k],
                    out_hbm.at[pl.ds((w * RG + rr) * RB + k * CH, CH)], sem3)

        def drain_stores(rr, st):
            for k in range(KG):
                pltpu.make_async_copy(
                    row_sets[st][k],
                    out_hbm.at[pl.ds((w * RG + rr) * RB + k * CH, CH)],
                    sem3).wait()

        fire_idx(0, 0)
        drain_idx(0, 0)
        fire_gathers(0)

        @pl.loop(0, RG, step=2)
        def _(r):
            for b in range(2):
                rr = r + b
                cur, nxt = b, 1 - b

                @pl.when(rr + 1 < RG)
                def _():
                    fire_idx(rr + 1, nxt)
                drain_gathers(cur)
                fire_stores(rr, cur)

                @pl.when(rr + 1 < RG)
                def _():
                    drain_idx(rr + 1, nxt)

                    @pl.when(rr >= 1)
                    def _():
                        drain_stores(rr - 1, nxt)
                    fire_gathers(nxt)

        drain_stores(RG - 2, 0)
        drain_stores(RG - 1, 1)


@functools.partial(
    pl.kernel,
    out_type=(jax.ShapeDtypeStruct((NP, DM), _f32),
              jax.ShapeDtypeStruct((NP, DM), _f32)),
    mesh=_sc_mesh(),
    compiler_params=pltpu.CompilerParams(use_tc_tiling_on_sc=False,
                                         needs_layout_passes=False),
    scratch_types=[pltpu.VMEM((KG, CH), jnp.int32),
                   pltpu.VMEM((KG * CH, 8), _f32),
                   pltpu.VMEM((KG * CH // 16, 16), _f32),
                   pltpu.SemaphoreType.DMA,
                   pltpu.VMEM_SHARED((NP, 8), _f32)],
)
def _scatter_k(gs_hbm, gd_hbm, e_hbm, d3_hbm, s3_hbm, z_hbm, mi_hbm, mo_hbm,
               idx_v, rows_v, e_v, sem, acc):
    c = lax.axis_index("c")
    t = lax.axis_index("s")

    def run(rows_hbm, idx_hbm, out_hbm):
        for h in range(3):
            pltpu.sync_copy(z_hbm.at[pl.ds(t * RPT, RPT)],
                            acc.at[pl.ds(t * RPT, RPT)])
            plsc.subcore_barrier()

            def round_body(r, carry, h=h):
                rid = t * RS + r
                pltpu.sync_copy(idx_hbm.at[rid], idx_v)
                pltpu.sync_copy(
                    rows_hbm.at[pl.ds(rid * RB, RB), pl.ds(h * 8, 8)], rows_v)
                pltpu.sync_copy(e_hbm.at[pl.ds(rid * (RB // 16), RB // 16)], e_v)

                def grp(g, cc):
                    rowi = g * 16 + lax.iota(jnp.int32, 16)
                    ev = e_v[g]
                    for j in range(8):
                        colj = jnp.full((16,), j, jnp.int32)
                        vals = plsc.load_gather(rows_v, [rowi, colj])
                        plsc.store_scatter(rows_v, [rowi, colj], vals * ev)
                    return cc
                lax.fori_loop(0, RB // 16, grp, 0)

                descs = [pltpu.async_copy(rows_v.at[pl.ds(k * CH, CH)],
                                          acc.at[idx_v.at[k]], sem, add=True)
                         for k in range(KG)]
                for dsc in descs:
                    dsc.wait()
                return carry
            lax.fori_loop(0, RS, round_body, 0)
            plsc.subcore_barrier()
            pltpu.sync_copy(acc.at[pl.ds(t * RPT, RPT)],
                            out_hbm.at[pl.ds(t * RPT, RPT), pl.ds(h * 8, 8)])
            plsc.subcore_barrier()

    @pl.when(c == 0)
    def _():
        run(gs_hbm, d3_hbm, mi_hbm)

    @pl.when(c == 1)
    def _():
        run(gd_hbm, s3_hbm, mo_hbm)


def kernel(x, edge_index, params):
    p = params
    el = p['emb_layers']
    eo = p['emb_out']
    ip = p['input_net']
    en = p['edge_net']
    nn = p['node_net']

    enc_ws = [el[0]['W'], _row(el[0]['b']), el[1]['W'], _row(el[1]['b']),
              el[2]['W'], _row(el[2]['b']), el[3]['W'], _row(el[3]['b']),
              eo['W'], _row(eo['b']),
              ip['W'], _row(ip['b']), _row(ip['g']), _row(ip['bn'])]
    I16 = jnp.eye(16, dtype=_f32)
    t16 = lambda v: jnp.tile(_row(v), (1, 16))
    M11 = jnp.kron(I16, jnp.full((11, 11), 1.0 / 11.0, _f32))
    edge_ws = [jnp.kron(I16, _pad20(en[0]['W'][:DC])),
               jnp.kron(I16, _pad20(en[0]['W'][DC:])),
               t16(en[0]['b']), t16(en[0]['g']), t16(en[0]['bn']), M11,
               jnp.kron(I16, en[1]['W']), t16(en[1]['b']), t16(en[1]['g']), t16(en[1]['bn']),
               jnp.kron(I16, en[2]['W']), t16(en[2]['b']), t16(en[2]['g']), t16(en[2]['bn']),
               jnp.kron(I16, en[3]['W']), t16(en[3]['b'])]
    node_ws = [_padw(nn[0]['W'][:DC], DM), _padw(nn[0]['W'][DC:2 * DC], DM),
               _pad20(nn[0]['W'][2 * DC:]),
               _row(nn[0]['b']), _row(nn[0]['g']), _row(nn[0]['bn']),
               nn[1]['W'], _row(nn[1]['b']), _row(nn[1]['g']), _row(nn[1]['bn']),
               nn[2]['W'], _row(nn[2]['b']), _row(nn[2]['g']), _row(nn[2]['bn']),
               nn[3]['W'], _row(nn[3]['b'])]

    s3 = edge_index[0].reshape(NR, KG, CH)
    d3 = edge_index[1].reshape(NR, KG, CH)
    zeros_np = jnp.zeros((NP, 8), _f32)

    spatial, xc = _encode(x, enc_ws)
    for _ in range(3):
        gs, gd = _gather_k(xc, s3, d3)
        gsp = gs.reshape(E16, 384)
        gdp = gd.reshape(E16, 384)
        e = _edge_pass(gsp, gdp, edge_ws, final=False)[0]
        mi, mo = _scatter_k(gs, gd, e, d3, s3, zeros_np)
        xc = _node_pass(mi, mo, xc, spatial, x, node_ws)[0]
    gs, gd = _gather_k(xc, s3, d3)
    logits = _edge_pass(gs.reshape(E16, 384), gd.reshape(E16, 384),
                        edge_ws, final=True)[0].reshape(E)

    return logits, spatial, edge_index, edge_index.shape[1] / x.shape[0]
